# baseline (speedup 1.0000x reference)
import functools

import jax
import jax.numpy as jnp
from jax import lax
from jax.experimental import pallas as pl
from jax.experimental.pallas import tpu as pltpu
from jax.experimental.pallas import tpu_sc as plsc

_L = 16
_NW = 32


def _sc_add(B, S, D):
    rows_per_w = S // _NW
    R = 16
    CH = R * D
    n_chunks = rows_per_w // R

    mesh = plsc.VectorSubcoreMesh(core_axis_name="c", subcore_axis_name="s")

    @functools.partial(
        pl.kernel,
        out_type=jax.ShapeDtypeStruct((B * S * D,), jnp.float32),
        scratch_types=[
            pltpu.VMEM((CH,), jnp.float32),
            pltpu.VMEM((CH,), jnp.float32),
        ],
        mesh=mesh,
    )
    def k(x_hbm, emb_hbm, out_hbm, xbuf, ebuf):
        wid = lax.axis_index("s") * 2 + lax.axis_index("c")
        base = wid * (rows_per_w * D)

        def chunk_body(ci, carry):
            eoff = base + ci * CH
            pltpu.sync_copy(emb_hbm.at[pl.ds(eoff, CH)], ebuf)

            def batch_body(b, carry2):
                off = b * (S * D) + eoff
                pltpu.sync_copy(x_hbm.at[pl.ds(off, CH)], xbuf)

                def vec_body(i, carry3):
                    for j in range(8):
                        sl = pl.ds((i * 8 + j) * _L, _L)
                        xbuf[sl] = xbuf[sl] + ebuf[sl]
                    return carry3

                lax.fori_loop(0, CH // (8 * _L), vec_body, 0)
                pltpu.sync_copy(xbuf, out_hbm.at[pl.ds(off, CH)])
                return carry2

            lax.fori_loop(0, B, batch_body, 0)
            return carry

        lax.fori_loop(0, n_chunks, chunk_body, 0)

    return k


def kernel(x, emb):
    B, S, D = x.shape
    out_flat = _sc_add(B, S, D)(x.reshape(-1), emb[:S].reshape(-1))
    return out_flat.reshape(B, S, D)


# speedup vs baseline: 1.2421x; 1.2421x over previous
import functools

import jax
import jax.numpy as jnp
from jax import lax
from jax.experimental import pallas as pl
from jax.experimental.pallas import tpu as pltpu
from jax.experimental.pallas import tpu_sc as plsc

_L = 16
_NW = 32
_UNROLL = 8


def _sc_add(B, S, D):
    rows_per_w = S // _NW
    R = 16
    n_chunks = rows_per_w // R
    n_steps = n_chunks * B

    mesh = plsc.VectorSubcoreMesh(core_axis_name="c", subcore_axis_name="s")

    @functools.partial(
        pl.kernel,
        out_type=jax.ShapeDtypeStruct((B * S, D), jnp.float32),
        scratch_types=[
            [pltpu.VMEM((R, D), jnp.float32) for _ in range(4)],
            [pltpu.VMEM((R, D), jnp.float32) for _ in range(2)],
            [pltpu.SemaphoreType.DMA for _ in range(4)],
            [pltpu.SemaphoreType.DMA for _ in range(4)],
            [pltpu.SemaphoreType.DMA for _ in range(2)],
        ],
        mesh=mesh,
    )
    def k(x_hbm, emb_hbm, out_hbm, xbufs, ebufs, sins, souts, seins):
        wid = lax.axis_index("s") * 2 + lax.axis_index("c")
        base = wid * rows_per_w

        def erow(c):
            return base + c * R

        def xrow(t):
            c, b = divmod(t, B)
            return b * S + erow(c)

        def start_xin(t):
            pltpu.async_copy(x_hbm.at[pl.ds(xrow(t), R)], xbufs[t % 4],
                             sins[t % 4])

        def wait_xin(t):
            pltpu.make_async_copy(x_hbm.at[pl.ds(xrow(t), R)], xbufs[t % 4],
                                  sins[t % 4]).wait()

        def start_xout(t):
            pltpu.async_copy(xbufs[t % 4], out_hbm.at[pl.ds(xrow(t), R)],
                             souts[t % 4])

        def wait_xout(t):
            pltpu.make_async_copy(xbufs[t % 4], out_hbm.at[pl.ds(xrow(t), R)],
                                  souts[t % 4]).wait()

        def start_ein(c):
            pltpu.async_copy(emb_hbm.at[pl.ds(erow(c), R)], ebufs[c % 2],
                             seins[c % 2])

        def wait_ein(c):
            pltpu.make_async_copy(emb_hbm.at[pl.ds(erow(c), R)], ebufs[c % 2],
                                  seins[c % 2]).wait()

        start_ein(0)
        start_xin(0)
        start_xin(1)

        for t in range(n_steps):
            c, b = divmod(t, B)
            if b == 0:
                wait_ein(c)
                if c + 1 < n_chunks:
                    start_ein(c + 1)
            wait_xin(t)
            if t + 2 < n_steps:
                if t - 2 >= 0:
                    wait_xout(t - 2)
                start_xin(t + 2)
            xbuf, ebuf = xbufs[t % 4], ebufs[c % 2]

            def row_body(r, carry, xbuf=xbuf, ebuf=ebuf):
                def vec_body(i, carry2):
                    for j in range(_UNROLL):
                        sl = pl.ds((i * _UNROLL + j) * _L, _L)
                        xbuf[r, sl] = xbuf[r, sl] + ebuf[r, sl]
                    return carry2

                lax.fori_loop(0, D // (_UNROLL * _L), vec_body, carry)
                return carry

            lax.fori_loop(0, R, row_body, 0)
            start_xout(t)

        for t in range(n_steps - 4, n_steps):
            wait_xout(t)

    return k


def kernel(x, emb):
    B, S, D = x.shape
    out2d = _sc_add(B, S, D)(x.reshape(B * S, D), emb[:S])
    return out2d.reshape(B, S, D)


# speedup vs baseline: 3.3797x; 2.7210x over previous
import functools

import jax
import jax.numpy as jnp
from jax import lax
from jax.experimental import pallas as pl
from jax.experimental.pallas import tpu as pltpu
from jax.experimental.pallas import tpu_sc as plsc

_L = 16
_NW = 32
_NSLOT = 3


def _sc_add(B, S, D):
    rows_per_w = S // _NW
    R = 8
    n_chunks = rows_per_w // R

    mesh = plsc.VectorSubcoreMesh(core_axis_name="c", subcore_axis_name="s")

    @functools.partial(
        pl.kernel,
        out_type=jax.ShapeDtypeStruct((B * S, D), jnp.float32),
        scratch_types=[
            [[pltpu.VMEM((R, D), jnp.float32) for _ in range(B)]
             for _ in range(_NSLOT)],
            [pltpu.VMEM((R, D), jnp.float32) for _ in range(_NSLOT)],
            [pltpu.SemaphoreType.DMA for _ in range(_NSLOT)],
            [pltpu.SemaphoreType.DMA for _ in range(_NSLOT)],
        ],
        mesh=mesh,
    )
    def k(x_hbm, emb_hbm, out_hbm, xslots, eslots, sins, souts):
        wid = lax.axis_index("s") * 2 + lax.axis_index("c")
        base = wid * rows_per_w

        def copies(c):
            sl = c % _NSLOT
            row = base + c * R
            ins = [(emb_hbm.at[pl.ds(row, R)], eslots[sl], sins[sl])]
            outs = []
            for b in range(B):
                ins.append((x_hbm.at[pl.ds(b * S + row, R)], xslots[sl][b],
                            sins[sl]))
                outs.append((xslots[sl][b], out_hbm.at[pl.ds(b * S + row, R)],
                             souts[sl]))
            return ins, outs

        def start_ins(c):
            for src, dst, sem in copies(c)[0]:
                pltpu.async_copy(src, dst, sem)

        def wait_ins(c):
            for src, dst, sem in copies(c)[0]:
                pltpu.make_async_copy(src, dst, sem).wait()

        def start_outs(c):
            for src, dst, sem in copies(c)[1]:
                pltpu.async_copy(src, dst, sem)

        def wait_outs(c):
            for src, dst, sem in copies(c)[1]:
                pltpu.make_async_copy(src, dst, sem).wait()

        for c in range(min(_NSLOT - 1, n_chunks)):
            start_ins(c)

        for t in range(n_chunks):
            if t + 1 < n_chunks and t + 1 >= _NSLOT - 1:
                if t >= 2:
                    wait_outs(t - 2)
                start_ins(t + 1)
            wait_ins(t)
            sl = t % _NSLOT
            xb, eb = xslots[sl], eslots[sl]

            def vec_body(i, carry, xb=xb, eb=eb):
                csl = pl.ds(i * _L, _L)
                for r in range(R):
                    ev = eb[r, csl]
                    for b in range(B):
                        xb[b][r, csl] = xb[b][r, csl] + ev
                return carry

            lax.fori_loop(0, D // _L, vec_body, 0)
            start_outs(t)

        for t in range(max(0, n_chunks - 3), n_chunks):
            wait_outs(t)

    return k


def kernel(x, emb):
    B, S, D = x.shape
    out2d = _sc_add(B, S, D)(x.reshape(B * S, D), emb[:S])
    return out2d.reshape(B, S, D)
